# Initial kernel scaffold; baseline (speedup 1.0000x reference)
import jax
import jax.numpy as jnp
from jax.experimental import pallas as pl


def kernel(x, edge_index, batch, params):
    raise NotImplementedError("write your pallas kernel here")


# speedup vs baseline: 29.2662x; 29.2662x over previous
import functools

import jax
import jax.numpy as jnp
from jax import lax
from jax.experimental import pallas as pl
from jax.experimental.pallas import tpu as pltpu
from jax.experimental.pallas import tpu_sc as plsc

N = 50000
K = 4
B = 64
IN = 4
H = 15
NP = 50176
Q = 128
C = 512
NBLK = NP // Q
NW = 32
BPW = NP // NW
GCH = 112
NCH = BPW // GCH
_BIG = 1e30


def _leaky(v):
    return jnp.where(v >= 0, v, 0.01 * v)


def _ce(ad, ai, bd, bi):
    take = ad <= bd
    lod = jnp.where(take, ad, bd)
    loi = jnp.where(take, ai, bi)
    hid = jnp.where(take, bd, ad)
    hii = jnp.where(take, bi, ai)
    return lod, loi, hid, hii


def _knn_body(t0_ref, nt_ref, qx_ref, qy_ref, qz_ref, px_ref, py_ref, pz_ref,
              bq_ref, bt_ref, o0, o1, o2, o3):
    g = pl.program_id(0)
    t0 = t0_ref[g]
    nt = nt_ref[g]
    qx = qx_ref[...]
    qy = qy_ref[...]
    qz = qz_ref[...]
    bq = bq_ref[...]
    qidx = g * Q + lax.broadcasted_iota(jnp.int32, (Q, 1), 0)

    def tile(t, carry):
        bd0, bi0, bd1, bi1, bd2, bi2, bd3, bi3 = carry
        cstart = (t0 + t) * C
        cx = px_ref[:, pl.ds(cstart, C)]
        cy = py_ref[:, pl.ds(cstart, C)]
        cz = pz_ref[:, pl.ds(cstart, C)]
        bc = bt_ref[:, pl.ds(cstart, C)]
        cidx = cstart + lax.broadcasted_iota(jnp.int32, (1, C), 1)
        dx = qx - cx
        d2 = dx * dx
        dy = qy - cy
        d2 = d2 + dy * dy
        dz = qz - cz
        d2 = d2 + dz * dz
        valid = (bq == bc) & (qidx != cidx) & (cidx < N)
        d2 = jnp.where(valid, d2, jnp.inf)
        cidx_f = cidx.astype(jnp.float32)
        tds, tis = [], []
        for _ in range(K):
            mk = jnp.min(d2, axis=1, keepdims=True)
            ik = jnp.min(jnp.where(d2 == mk, cidx_f, _BIG), axis=1,
                         keepdims=True)
            tds.append(mk)
            tis.append(ik)
            d2 = jnp.where(cidx_f == ik, jnp.inf, d2)
        l0d, l0i, _, _ = _ce(bd0, bi0, tds[3], tis[3])
        l1d, l1i, _, _ = _ce(bd1, bi1, tds[2], tis[2])
        l2d, l2i, _, _ = _ce(bd2, bi2, tds[1], tis[1])
        l3d, l3i, _, _ = _ce(bd3, bi3, tds[0], tis[0])
        l0d, l0i, l2d, l2i = _ce(l0d, l0i, l2d, l2i)
        l1d, l1i, l3d, l3i = _ce(l1d, l1i, l3d, l3i)
        l0d, l0i, l1d, l1i = _ce(l0d, l0i, l1d, l1i)
        l2d, l2i, l3d, l3i = _ce(l2d, l2i, l3d, l3i)
        return l0d, l0i, l1d, l1i, l2d, l2i, l3d, l3i

    inf = jnp.full((Q, 1), jnp.inf, jnp.float32)
    zero = jnp.zeros((Q, 1), jnp.float32)
    carry = (inf, zero, inf, zero, inf, zero, inf, zero)
    carry = lax.fori_loop(0, nt, tile, carry)
    _, i0, _, i1, _, i2, _, i3 = carry
    o0[...] = i0.astype(jnp.int32)
    o1[...] = i1.astype(jnp.int32)
    o2[...] = i2.astype(jnp.int32)
    o3[...] = i3.astype(jnp.int32)


def _knn(pos_pad, batch_pad, t0_arr, nt_arr):
    qx = pos_pad[:, 0:1]
    qy = pos_pad[:, 1:2]
    qz = pos_pad[:, 2:3]
    px = pos_pad[:, 0].reshape(1, NP)
    py = pos_pad[:, 1].reshape(1, NP)
    pz = pos_pad[:, 2].reshape(1, NP)
    bq = batch_pad.reshape(NP, 1)
    bt = batch_pad.reshape(1, NP)
    grid_spec = pltpu.PrefetchScalarGridSpec(
        num_scalar_prefetch=2,
        grid=(NBLK,),
        in_specs=[
            pl.BlockSpec((Q, 1), lambda i, s0, s1: (i, 0)),
            pl.BlockSpec((Q, 1), lambda i, s0, s1: (i, 0)),
            pl.BlockSpec((Q, 1), lambda i, s0, s1: (i, 0)),
            pl.BlockSpec((1, NP), lambda i, s0, s1: (0, 0)),
            pl.BlockSpec((1, NP), lambda i, s0, s1: (0, 0)),
            pl.BlockSpec((1, NP), lambda i, s0, s1: (0, 0)),
            pl.BlockSpec((Q, 1), lambda i, s0, s1: (i, 0)),
            pl.BlockSpec((1, NP), lambda i, s0, s1: (0, 0)),
        ],
        out_specs=[pl.BlockSpec((Q, 1), lambda i, s0, s1: (i, 0))] * 4,
    )
    return pl.pallas_call(
        _knn_body,
        grid_spec=grid_spec,
        out_shape=[jax.ShapeDtypeStruct((NP, 1), jnp.int32)] * 4,
    )(t0_arr, nt_arr, qx, qy, qz, px, py, pz, bq, bt)


def _gather_sum(table, idx4):
    mesh = plsc.VectorSubcoreMesh(core_axis_name="c", subcore_axis_name="s")

    @functools.partial(
        pl.kernel,
        mesh=mesh,
        compiler_params=pltpu.CompilerParams(use_tc_tiling_on_sc=False),
        out_type=jax.ShapeDtypeStruct((NP, 16), jnp.float32),
        scratch_types=(
            [pltpu.VMEM((GCH,), jnp.int32) for _ in range(K)]
            + [pltpu.VMEM((GCH, 16), jnp.float32) for _ in range(K)]
            + [pltpu.SemaphoreType.DMA]
        ),
    )
    def gsum(n0, n1, n2, n3, tab_hbm, out_hbm, i0, i1, i2, i3,
             r0, r1, r2, r3, sem):
        nv = (n0, n1, n2, n3)
        iv = (i0, i1, i2, i3)
        rv = (r0, r1, r2, r3)
        wid = lax.axis_index("s") * 2 + lax.axis_index("c")
        base = wid * BPW

        def chunk(c, _):
            off = base + c * GCH
            for k in range(K):
                pltpu.sync_copy(nv[k].at[pl.ds(off, GCH)], iv[k])
            copies = [pltpu.async_copy(tab_hbm.at[iv[k]], rv[k], sem)
                      for k in range(K)]
            for cp in copies:
                cp.wait()

            def rowadd(j, _):
                r0[j, :] = ((r0[j, :] + r1[j, :]) + (r2[j, :] + r3[j, :]))
                return 0

            lax.fori_loop(0, GCH, rowadd, 0)
            pltpu.sync_copy(r0, out_hbm.at[pl.ds(off, GCH)])
            return 0

        lax.fori_loop(0, NCH, chunk, 0)

    return gsum(idx4[0], idx4[1], idx4[2], idx4[3], table)


def _dot(a, b):
    return jnp.dot(a, b, preferred_element_type=jnp.float32)


def _gru_t(hT, aT, gate_w):
    (wih_r, wih_z, wih_n, whh_r, whh_z, whh_n,
     bih_r, bih_z, bih_n, bhh_r, bhh_z, bhh_n) = gate_w
    ir = _dot(wih_r, aT) + bih_r
    iz = _dot(wih_z, aT) + bih_z
    inn = _dot(wih_n, aT) + bih_n
    hr = _dot(whh_r, hT) + bhh_r
    hz = _dot(whh_z, hT) + bhh_z
    hn = _dot(whh_n, hT) + bhh_n
    r = jax.nn.sigmoid(ir + hr)
    z = jax.nn.sigmoid(iz + hz)
    n = jnp.tanh(inn + r * hn)
    return (1.0 - z) * n + z * hT


def _bn_t(vT, g, b, maskN):
    msum = jnp.sum(jnp.where(maskN, vT, 0.0), axis=1, keepdims=True)
    m = msum / N
    dv = vT - m
    var = jnp.sum(jnp.where(maskN, dv * dv, 0.0), axis=1, keepdims=True) / N
    return dv / jnp.sqrt(var + 1e-5) * g + b


def _res_t(vT, pr, maskN):
    bn1_g, bn1_b, l1_w, l1_b, bn2_g, bn2_b, l2_w, l2_b = pr
    h = _bn_t(vT, bn1_g, bn1_b, maskN)
    h = _leaky(h)
    h = _dot(l1_w, h) + l1_b
    h = _bn_t(h, bn2_g, bn2_b, maskN)
    h = _leaky(h)
    h = _dot(l2_w, h) + l2_b
    return _leaky(h)


def _pre_body(xT_ref, wT_ref, o_ref):
    o_ref[...] = _dot(wT_ref[...], xT_ref[...])


def _mid_body(xT_ref, aggT_ref, *refs):
    gates = refs[0:12]
    (nn1_w, nn1_b,
     r1a, r1b, r1c, r1d, r1e, r1f, r1g, r1h,
     r2a, r2b, r2c, r2d, r2e, r2f, r2g, r2h,
     nn2_w, nn2_b, gg2_wT, h2_ref, m2_ref) = refs[12:]
    maskN = lax.broadcasted_iota(jnp.int32, (1, NP), 1) < N
    xT = xT_ref[...]
    aggT = aggT_ref[...]
    h = _gru_t(xT, aggT, [g[...] for g in gates])
    h = _leaky(h)
    h = _leaky(_dot(nn1_w[...], h) + nn1_b[...])
    h = h + _res_t(h, [r[...] for r in (r1a, r1b, r1c, r1d, r1e, r1f, r1g, r1h)],
                   maskN)
    h = h + _res_t(h, [r[...] for r in (r2a, r2b, r2c, r2d, r2e, r2f, r2g, r2h)],
                   maskN)
    h = _leaky(_dot(nn2_w[...], h) + nn2_b[...])
    h2_ref[...] = h
    m2_ref[...] = _dot(gg2_wT[...], h)


def _post_body(hT_ref, aggT_ref, bt_ref, *refs):
    gates = refs[0:12]
    (r3a, r3b, r3c, r3d, r3e, r3f, r3g, r3h,
     r4a, r4b, r4c, r4d, r4e, r4f, r4g, r4h,
     nn3_w, nn3_b, ncat_wT, ncat_b, out_ref) = refs[12:]
    maskN = lax.broadcasted_iota(jnp.int32, (1, NP), 1) < N
    hT = hT_ref[...]
    aggT = aggT_ref[...]
    h = _gru_t(hT, aggT, [g[...] for g in gates])
    h = _leaky(h)
    h = h + _res_t(h, [r[...] for r in (r3a, r3b, r3c, r3d, r3e, r3f, r3g, r3h)],
                   maskN)
    h = h + _res_t(h, [r[...] for r in (r4a, r4b, r4c, r4d, r4e, r4f, r4g, r4h)],
                   maskN)
    hf = _leaky(_dot(nn3_w[...], h) + nn3_b[...])
    bt = bt_ref[...]
    bvals = lax.broadcasted_iota(jnp.int32, (B, 1), 0)
    CH = NP // 8
    amax = jnp.full((B, 1), -jnp.inf, jnp.float32)
    amin = jnp.full((B, 1), jnp.inf, jnp.float32)
    asum = jnp.zeros((B, 1), jnp.float32)
    acnt = jnp.zeros((B, 1), jnp.float32)
    for ci in range(8):
        hc = hf[:, ci * CH:(ci + 1) * CH]
        bc = bt[:, ci * CH:(ci + 1) * CH]
        mask = bc == bvals
        amax = jnp.maximum(amax, jnp.max(jnp.where(mask, hc, -jnp.inf),
                                         axis=1, keepdims=True))
        amin = jnp.minimum(amin, jnp.min(jnp.where(mask, hc, jnp.inf),
                                         axis=1, keepdims=True))
        asum = asum + jnp.sum(jnp.where(mask, hc, 0.0), axis=1, keepdims=True)
        acnt = acnt + jnp.sum(mask.astype(jnp.float32), axis=1, keepdims=True)
    amean = asum / jnp.maximum(acnt, 1.0)
    g = jnp.concatenate([amax, amin, asum, amean], axis=1)
    out_ref[...] = _leaky(_dot(g, ncat_wT[...]) + ncat_b[...])


def _full_vmem_call(body, args, out_shape):
    return pl.pallas_call(body, out_shape=out_shape)(*args)


def kernel(x, edge_index, batch, params):
    del edge_index
    p = params
    f32 = jnp.float32

    pos_pad = jnp.pad(x[:, :3], ((0, NP - N), (0, 0)))
    batch_pad = jnp.pad(batch, (0, NP - N), constant_values=63)
    batch_pool = jnp.pad(batch, (0, NP - N), constant_values=9999)

    bvals = jnp.arange(B, dtype=jnp.int32)
    seg_lo = jnp.searchsorted(batch, bvals, side="left").astype(jnp.int32)
    seg_hi = jnp.searchsorted(batch, bvals, side="right").astype(jnp.int32)
    qs = jnp.minimum(jnp.arange(NBLK, dtype=jnp.int32) * Q, N - 1)
    ql = jnp.minimum(jnp.arange(NBLK, dtype=jnp.int32) * Q + (Q - 1), N - 1)
    lo = seg_lo[batch[qs]]
    hi = seg_hi[batch[ql]]
    t0_arr = lo // C
    nt_arr = (hi + (C - 1)) // C - t0_arr

    nbr = _knn(pos_pad, batch_pad, t0_arr, nt_arr)
    idx4 = [n.reshape(NP) for n in nbr]

    xT = jnp.pad(x, ((0, NP - N), (0, 0))).T
    m1T = pl.pallas_call(
        _pre_body, out_shape=jax.ShapeDtypeStruct((IN, NP), f32),
    )(xT, p["gg1_w"].T)
    m1tab = jnp.pad(m1T.T, ((0, 0), (0, 16 - IN)))
    agg1 = _gather_sum(m1tab, idx4)
    agg1T = agg1[:, :IN].T

    def cvec(v):
        return v.reshape(-1, 1).astype(f32)

    def res_params(name):
        return [p[name + "_bn1_g"].reshape(-1, 1), cvec(p[name + "_bn1_b"]),
                p[name + "_l1_w"], cvec(p[name + "_l1_b"]),
                p[name + "_bn2_g"].reshape(-1, 1), cvec(p[name + "_bn2_b"]),
                p[name + "_l2_w"], cvec(p[name + "_l2_b"])]

    def gate_params(prefix):
        wih, whh = p[prefix + "_wih"], p[prefix + "_whh"]
        bih, bhh = p[prefix + "_bih"], p[prefix + "_bhh"]
        f = wih.shape[1]
        return ([wih[0:f], wih[f:2 * f], wih[2 * f:3 * f],
                 whh[0:f], whh[f:2 * f], whh[2 * f:3 * f]]
                + [cvec(bih[0:f]), cvec(bih[f:2 * f]), cvec(bih[2 * f:3 * f]),
                   cvec(bhh[0:f]), cvec(bhh[f:2 * f]), cvec(bhh[2 * f:3 * f])])

    mid_args = ([xT, agg1T] + gate_params("gg1")
                + [p["nn1_w"], cvec(p["nn1_b"])]
                + res_params("r1") + res_params("r2")
                + [p["nn2_w"], cvec(p["nn2_b"]), p["gg2_w"].T])
    h2T, m2T = _full_vmem_call(_mid_body, mid_args,
                               [jax.ShapeDtypeStruct((H, NP), f32),
                                jax.ShapeDtypeStruct((H, NP), f32)])

    m2tab = jnp.pad(m2T.T, ((0, 0), (0, 16 - H)))
    agg2 = _gather_sum(m2tab, idx4)
    agg2T = agg2[:, :H].T

    post_args = ([h2T, agg2T, batch_pool.reshape(1, NP)]
                 + gate_params("gg2")
                 + res_params("r3") + res_params("r4")
                 + [p["nn3_w"], cvec(p["nn3_b"]),
                    p["ncat_w"].T, cvec(p["ncat_b"]).T])
    out = _full_vmem_call(_post_body, post_args,
                          jax.ShapeDtypeStruct((B, 1), f32))
    return out


# speedup vs baseline: 33.6758x; 1.1507x over previous
import functools

import jax
import jax.numpy as jnp
from jax import lax
from jax.experimental import pallas as pl
from jax.experimental.pallas import tpu as pltpu
from jax.experimental.pallas import tpu_sc as plsc

N = 50000
K = 4
B = 64
IN = 4
H = 15
NP = 50176
Q = 128
C = 1024
NBLK = NP // Q
NW = 32
BPW = NP // NW
GCH = 112
NCH = BPW // GCH
_BIG = 1e30


def _leaky(v):
    return jnp.where(v >= 0, v, 0.01 * v)


def _ce(ad, ai, bd, bi):
    take = ad <= bd
    lod = jnp.where(take, ad, bd)
    loi = jnp.where(take, ai, bi)
    hid = jnp.where(take, bd, ad)
    hii = jnp.where(take, bi, ai)
    return lod, loi, hid, hii


def _knn_body(t0_ref, nt_ref, qx_ref, qy_ref, qz_ref, px_ref, py_ref, pz_ref,
              bq_ref, bt_ref, o0, o1, o2, o3):
    g = pl.program_id(0)
    t0 = t0_ref[g]
    nt = nt_ref[g]
    qx = qx_ref[...]
    qy = qy_ref[...]
    qz = qz_ref[...]
    bq = bq_ref[...]
    qidx = g * Q + lax.broadcasted_iota(jnp.int32, (Q, 1), 0)

    def tile(t, carry):
        bd0, bi0, bd1, bi1, bd2, bi2, bd3, bi3 = carry
        cstart = (t0 + t) * C
        cx = px_ref[:, pl.ds(cstart, C)]
        cy = py_ref[:, pl.ds(cstart, C)]
        cz = pz_ref[:, pl.ds(cstart, C)]
        bc = bt_ref[:, pl.ds(cstart, C)]
        cidx = cstart + lax.broadcasted_iota(jnp.int32, (1, C), 1)
        dx = qx - cx
        d2 = dx * dx
        dy = qy - cy
        d2 = d2 + dy * dy
        dz = qz - cz
        d2 = d2 + dz * dz
        valid = (bq == bc) & (qidx != cidx) & (cidx < N)
        d2 = jnp.where(valid, d2, jnp.inf)
        cidx_f = cidx.astype(jnp.float32)
        tds, tis = [], []
        for _ in range(K):
            mk = jnp.min(d2, axis=1, keepdims=True)
            ik = jnp.min(jnp.where(d2 == mk, cidx_f, _BIG), axis=1,
                         keepdims=True)
            tds.append(mk)
            tis.append(ik)
            d2 = jnp.where(cidx_f == ik, jnp.inf, d2)
        l0d, l0i, _, _ = _ce(bd0, bi0, tds[3], tis[3])
        l1d, l1i, _, _ = _ce(bd1, bi1, tds[2], tis[2])
        l2d, l2i, _, _ = _ce(bd2, bi2, tds[1], tis[1])
        l3d, l3i, _, _ = _ce(bd3, bi3, tds[0], tis[0])
        l0d, l0i, l2d, l2i = _ce(l0d, l0i, l2d, l2i)
        l1d, l1i, l3d, l3i = _ce(l1d, l1i, l3d, l3i)
        l0d, l0i, l1d, l1i = _ce(l0d, l0i, l1d, l1i)
        l2d, l2i, l3d, l3i = _ce(l2d, l2i, l3d, l3i)
        return l0d, l0i, l1d, l1i, l2d, l2i, l3d, l3i

    inf = jnp.full((Q, 1), jnp.inf, jnp.float32)
    zero = jnp.zeros((Q, 1), jnp.float32)
    carry = (inf, zero, inf, zero, inf, zero, inf, zero)
    carry = lax.fori_loop(0, nt, tile, carry)
    _, i0, _, i1, _, i2, _, i3 = carry
    o0[...] = i0.astype(jnp.int32)
    o1[...] = i1.astype(jnp.int32)
    o2[...] = i2.astype(jnp.int32)
    o3[...] = i3.astype(jnp.int32)


def _knn(pos_pad, batch_pad, t0_arr, nt_arr):
    qx = pos_pad[:, 0:1]
    qy = pos_pad[:, 1:2]
    qz = pos_pad[:, 2:3]
    px = pos_pad[:, 0].reshape(1, NP)
    py = pos_pad[:, 1].reshape(1, NP)
    pz = pos_pad[:, 2].reshape(1, NP)
    bq = batch_pad.reshape(NP, 1)
    bt = batch_pad.reshape(1, NP)
    grid_spec = pltpu.PrefetchScalarGridSpec(
        num_scalar_prefetch=2,
        grid=(NBLK,),
        in_specs=[
            pl.BlockSpec((Q, 1), lambda i, s0, s1: (i, 0)),
            pl.BlockSpec((Q, 1), lambda i, s0, s1: (i, 0)),
            pl.BlockSpec((Q, 1), lambda i, s0, s1: (i, 0)),
            pl.BlockSpec((1, NP), lambda i, s0, s1: (0, 0)),
            pl.BlockSpec((1, NP), lambda i, s0, s1: (0, 0)),
            pl.BlockSpec((1, NP), lambda i, s0, s1: (0, 0)),
            pl.BlockSpec((Q, 1), lambda i, s0, s1: (i, 0)),
            pl.BlockSpec((1, NP), lambda i, s0, s1: (0, 0)),
        ],
        out_specs=[pl.BlockSpec((Q, 1), lambda i, s0, s1: (i, 0))] * 4,
    )
    return pl.pallas_call(
        _knn_body,
        grid_spec=grid_spec,
        out_shape=[jax.ShapeDtypeStruct((NP, 1), jnp.int32)] * 4,
    )(t0_arr, nt_arr, qx, qy, qz, px, py, pz, bq, bt)


def _gather_sum(table, idx4):
    mesh = plsc.VectorSubcoreMesh(core_axis_name="c", subcore_axis_name="s")

    @functools.partial(
        pl.kernel,
        mesh=mesh,
        compiler_params=pltpu.CompilerParams(use_tc_tiling_on_sc=False),
        out_type=jax.ShapeDtypeStruct((NP, 16), jnp.float32),
        scratch_types=(
            [pltpu.VMEM((GCH,), jnp.int32) for _ in range(K)]
            + [pltpu.VMEM((GCH, 16), jnp.float32) for _ in range(K)]
            + [pltpu.SemaphoreType.DMA]
        ),
    )
    def gsum(n0, n1, n2, n3, tab_hbm, out_hbm, i0, i1, i2, i3,
             r0, r1, r2, r3, sem):
        nv = (n0, n1, n2, n3)
        iv = (i0, i1, i2, i3)
        rv = (r0, r1, r2, r3)
        wid = lax.axis_index("s") * 2 + lax.axis_index("c")
        base = wid * BPW

        def chunk(c, _):
            off = base + c * GCH
            for k in range(K):
                pltpu.sync_copy(nv[k].at[pl.ds(off, GCH)], iv[k])
            copies = [pltpu.async_copy(tab_hbm.at[iv[k]], rv[k], sem)
                      for k in range(K)]
            for cp in copies:
                cp.wait()

            def rowadd(j, _):
                r0[j, :] = ((r0[j, :] + r1[j, :]) + (r2[j, :] + r3[j, :]))
                return 0

            lax.fori_loop(0, GCH, rowadd, 0)
            pltpu.sync_copy(r0, out_hbm.at[pl.ds(off, GCH)])
            return 0

        lax.fori_loop(0, NCH, chunk, 0)

    return gsum(idx4[0], idx4[1], idx4[2], idx4[3], table)


def _dot(a, b):
    return jnp.dot(a, b, preferred_element_type=jnp.float32)


def _gru_t(hT, aT, gate_w):
    (wih_r, wih_z, wih_n, whh_r, whh_z, whh_n,
     bih_r, bih_z, bih_n, bhh_r, bhh_z, bhh_n) = gate_w
    ir = _dot(wih_r, aT) + bih_r
    iz = _dot(wih_z, aT) + bih_z
    inn = _dot(wih_n, aT) + bih_n
    hr = _dot(whh_r, hT) + bhh_r
    hz = _dot(whh_z, hT) + bhh_z
    hn = _dot(whh_n, hT) + bhh_n
    r = jax.nn.sigmoid(ir + hr)
    z = jax.nn.sigmoid(iz + hz)
    n = jnp.tanh(inn + r * hn)
    return (1.0 - z) * n + z * hT


def _bn_t(vT, g, b, maskN):
    msum = jnp.sum(jnp.where(maskN, vT, 0.0), axis=1, keepdims=True)
    m = msum / N
    dv = vT - m
    var = jnp.sum(jnp.where(maskN, dv * dv, 0.0), axis=1, keepdims=True) / N
    return dv / jnp.sqrt(var + 1e-5) * g + b


def _res_t(vT, pr, maskN):
    bn1_g, bn1_b, l1_w, l1_b, bn2_g, bn2_b, l2_w, l2_b = pr
    h = _bn_t(vT, bn1_g, bn1_b, maskN)
    h = _leaky(h)
    h = _dot(l1_w, h) + l1_b
    h = _bn_t(h, bn2_g, bn2_b, maskN)
    h = _leaky(h)
    h = _dot(l2_w, h) + l2_b
    return _leaky(h)


def _pre_body(xT_ref, wT_ref, o_ref):
    o_ref[...] = _dot(wT_ref[...], xT_ref[...])


def _mid_body(xT_ref, aggT_ref, *refs):
    gates = refs[0:12]
    (nn1_w, nn1_b,
     r1a, r1b, r1c, r1d, r1e, r1f, r1g, r1h,
     r2a, r2b, r2c, r2d, r2e, r2f, r2g, r2h,
     nn2_w, nn2_b, gg2_wT, h2_ref, m2_ref) = refs[12:]
    maskN = lax.broadcasted_iota(jnp.int32, (1, NP), 1) < N
    xT = xT_ref[...]
    aggT = aggT_ref[...]
    h = _gru_t(xT, aggT, [g[...] for g in gates])
    h = _leaky(h)
    h = _leaky(_dot(nn1_w[...], h) + nn1_b[...])
    h = h + _res_t(h, [r[...] for r in (r1a, r1b, r1c, r1d, r1e, r1f, r1g, r1h)],
                   maskN)
    h = h + _res_t(h, [r[...] for r in (r2a, r2b, r2c, r2d, r2e, r2f, r2g, r2h)],
                   maskN)
    h = _leaky(_dot(nn2_w[...], h) + nn2_b[...])
    h2_ref[...] = h
    m2_ref[...] = _dot(gg2_wT[...], h)


def _post_body(hT_ref, aggT_ref, bt_ref, *refs):
    gates = refs[0:12]
    (r3a, r3b, r3c, r3d, r3e, r3f, r3g, r3h,
     r4a, r4b, r4c, r4d, r4e, r4f, r4g, r4h,
     nn3_w, nn3_b, ncat_wT, ncat_b, out_ref) = refs[12:]
    maskN = lax.broadcasted_iota(jnp.int32, (1, NP), 1) < N
    hT = hT_ref[...]
    aggT = aggT_ref[...]
    h = _gru_t(hT, aggT, [g[...] for g in gates])
    h = _leaky(h)
    h = h + _res_t(h, [r[...] for r in (r3a, r3b, r3c, r3d, r3e, r3f, r3g, r3h)],
                   maskN)
    h = h + _res_t(h, [r[...] for r in (r4a, r4b, r4c, r4d, r4e, r4f, r4g, r4h)],
                   maskN)
    hf = _leaky(_dot(nn3_w[...], h) + nn3_b[...])
    bt = bt_ref[...]
    bvals = lax.broadcasted_iota(jnp.int32, (B, 1), 0)
    CH = NP // 8
    amax = jnp.full((B, 1), -jnp.inf, jnp.float32)
    amin = jnp.full((B, 1), jnp.inf, jnp.float32)
    asum = jnp.zeros((B, 1), jnp.float32)
    acnt = jnp.zeros((B, 1), jnp.float32)
    for ci in range(8):
        hc = hf[:, ci * CH:(ci + 1) * CH]
        bc = bt[:, ci * CH:(ci + 1) * CH]
        mask = bc == bvals
        amax = jnp.maximum(amax, jnp.max(jnp.where(mask, hc, -jnp.inf),
                                         axis=1, keepdims=True))
        amin = jnp.minimum(amin, jnp.min(jnp.where(mask, hc, jnp.inf),
                                         axis=1, keepdims=True))
        asum = asum + jnp.sum(jnp.where(mask, hc, 0.0), axis=1, keepdims=True)
        acnt = acnt + jnp.sum(mask.astype(jnp.float32), axis=1, keepdims=True)
    amean = asum / jnp.maximum(acnt, 1.0)
    g = jnp.concatenate([amax, amin, asum, amean], axis=1)
    out_ref[...] = _leaky(_dot(g, ncat_wT[...]) + ncat_b[...])


def _full_vmem_call(body, args, out_shape):
    return pl.pallas_call(body, out_shape=out_shape)(*args)


def kernel(x, edge_index, batch, params):
    del edge_index
    p = params
    f32 = jnp.float32

    pos_pad = jnp.pad(x[:, :3], ((0, NP - N), (0, 0)))
    batch_pad = jnp.pad(batch, (0, NP - N), constant_values=63)
    batch_pool = jnp.pad(batch, (0, NP - N), constant_values=9999)

    bvals = jnp.arange(B, dtype=jnp.int32)
    seg_lo = jnp.searchsorted(batch, bvals, side="left").astype(jnp.int32)
    seg_hi = jnp.searchsorted(batch, bvals, side="right").astype(jnp.int32)
    qs = jnp.minimum(jnp.arange(NBLK, dtype=jnp.int32) * Q, N - 1)
    ql = jnp.minimum(jnp.arange(NBLK, dtype=jnp.int32) * Q + (Q - 1), N - 1)
    lo = seg_lo[batch[qs]]
    hi = seg_hi[batch[ql]]
    t0_arr = lo // C
    nt_arr = (hi + (C - 1)) // C - t0_arr

    nbr = _knn(pos_pad, batch_pad, t0_arr, nt_arr)
    idx4 = [n.reshape(NP) for n in nbr]

    xT = jnp.pad(x, ((0, NP - N), (0, 0))).T
    m1T = pl.pallas_call(
        _pre_body, out_shape=jax.ShapeDtypeStruct((IN, NP), f32),
    )(xT, p["gg1_w"].T)
    m1tab = jnp.pad(m1T.T, ((0, 0), (0, 16 - IN)))
    agg1 = _gather_sum(m1tab, idx4)
    agg1T = agg1[:, :IN].T

    def cvec(v):
        return v.reshape(-1, 1).astype(f32)

    def res_params(name):
        return [p[name + "_bn1_g"].reshape(-1, 1), cvec(p[name + "_bn1_b"]),
                p[name + "_l1_w"], cvec(p[name + "_l1_b"]),
                p[name + "_bn2_g"].reshape(-1, 1), cvec(p[name + "_bn2_b"]),
                p[name + "_l2_w"], cvec(p[name + "_l2_b"])]

    def gate_params(prefix):
        wih, whh = p[prefix + "_wih"], p[prefix + "_whh"]
        bih, bhh = p[prefix + "_bih"], p[prefix + "_bhh"]
        f = wih.shape[1]
        return ([wih[0:f], wih[f:2 * f], wih[2 * f:3 * f],
                 whh[0:f], whh[f:2 * f], whh[2 * f:3 * f]]
                + [cvec(bih[0:f]), cvec(bih[f:2 * f]), cvec(bih[2 * f:3 * f]),
                   cvec(bhh[0:f]), cvec(bhh[f:2 * f]), cvec(bhh[2 * f:3 * f])])

    mid_args = ([xT, agg1T] + gate_params("gg1")
                + [p["nn1_w"], cvec(p["nn1_b"])]
                + res_params("r1") + res_params("r2")
                + [p["nn2_w"], cvec(p["nn2_b"]), p["gg2_w"].T])
    h2T, m2T = _full_vmem_call(_mid_body, mid_args,
                               [jax.ShapeDtypeStruct((H, NP), f32),
                                jax.ShapeDtypeStruct((H, NP), f32)])

    m2tab = jnp.pad(m2T.T, ((0, 0), (0, 16 - H)))
    agg2 = _gather_sum(m2tab, idx4)
    agg2T = agg2[:, :H].T

    post_args = ([h2T, agg2T, batch_pool.reshape(1, NP)]
                 + gate_params("gg2")
                 + res_params("r3") + res_params("r4")
                 + [p["nn3_w"], cvec(p["nn3_b"]),
                    p["ncat_w"].T, cvec(p["ncat_b"]).T])
    out = _full_vmem_call(_post_body, post_args,
                          jax.ShapeDtypeStruct((B, 1), f32))
    return out
